# baseline (speedup 1.0000x reference)
import functools

import jax
import jax.numpy as jnp
from jax import lax
from jax.experimental import pallas as pl
from jax.experimental.pallas import tpu as pltpu
from jax.experimental.pallas import tpu_sc as plsc

BT, SQ, NF = 1024, 50, 12
ROWS = BT * SQ
WIDTH = 782
NCORES, NSUB, LANES = 2, 16, 16
NW = NCORES * NSUB
RPW = ROWS // NW
RB = 32
NB = RPW // RB
GPB = RB // LANES

SEG_OFF = (0, 32, 59, 187, 315, 342, 369, 396, 524, 652)
COST_COL = (780, 781)


def _scatter_group(feat, obuf, rows, write):
    fbase = rows * NF
    obase = (rows % RB) * WIDTH
    ones = jnp.ones((LANES,), jnp.float32)
    zeros = jnp.zeros((LANES,), jnp.float32)
    for k, off in enumerate(SEG_OFF):
        col = plsc.load_gather(feat, [fbase + k]).astype(jnp.int32) + off
        plsc.store_scatter(obuf, [obase + col], ones if write else zeros)
    for j, col in enumerate(COST_COL):
        if write:
            val = plsc.load_gather(feat, [fbase + (10 + j)])
        else:
            val = zeros
        plsc.store_scatter(obuf, [obase + col], val)


def _sc_body(feat_hbm, out_hbm, feat, buf0, buf1, sem0, sem1):
    wid = lax.axis_index("s") * NCORES + lax.axis_index("c")
    row0 = wid * RPW
    iota = lax.broadcasted_iota(jnp.int32, (LANES,), 0)
    bufs = (buf0, buf1)
    sems = (sem0, sem1)

    pltpu.sync_copy(feat_hbm.at[pl.ds(row0 * NF, RPW * NF)], feat)
    zeros = jnp.zeros((LANES,), jnp.float32)

    def _zero(i, _):
        buf0[pl.ds(i * LANES, LANES)] = zeros
        buf1[pl.ds(i * LANES, LANES)] = zeros
        return 0

    lax.fori_loop(0, RB * WIDTH // LANES, _zero, 0)

    def _out_slice(bi):
        start = pl.multiple_of((row0 + bi * RB) * WIDTH, 8)
        return out_hbm.at[pl.ds(start, RB * WIDTH)]

    def _run_batch(bi, b, first):
        if not first:
            pltpu.make_async_copy(bufs[b], _out_slice(bi), sems[b]).wait()
            for g in range(GPB):
                rows = (bi - 2) * RB + g * LANES + iota
                _scatter_group(feat, bufs[b], rows, write=False)
        for g in range(GPB):
            rows = bi * RB + g * LANES + iota
            _scatter_group(feat, bufs[b], rows, write=True)
        pltpu.async_copy(bufs[b], _out_slice(bi), sems[b])

    _run_batch(0, 0, True)
    _run_batch(1, 1, True)

    def _pair(i, _):
        _run_batch(i * 2, 0, False)
        _run_batch(i * 2 + 1, 1, False)
        return 0

    lax.fori_loop(1, NB // 2, _pair, 0)

    pltpu.make_async_copy(buf0, _out_slice(NB - 2), sem0).wait()
    pltpu.make_async_copy(buf1, _out_slice(NB - 1), sem1).wait()


@jax.jit
def _onehot_embed(feature_flat):
    mesh = plsc.VectorSubcoreMesh(
        core_axis_name="c", subcore_axis_name="s",
        num_cores=NCORES, num_subcores=NSUB)
    run = pl.kernel(
        _sc_body,
        out_type=jax.ShapeDtypeStruct((ROWS * WIDTH,), jnp.float32),
        mesh=mesh,
        compiler_params=pltpu.CompilerParams(needs_layout_passes=False),
        scratch_types=[
            pltpu.VMEM((RPW * NF,), jnp.float32),
            pltpu.VMEM((RB * WIDTH,), jnp.float32),
            pltpu.VMEM((RB * WIDTH,), jnp.float32),
            pltpu.SemaphoreType.DMA,
            pltpu.SemaphoreType.DMA,
        ],
    )
    return run(feature_flat)


def kernel(feature):
    out = _onehot_embed(feature.reshape(ROWS * NF))
    return out.reshape(BT, SQ, WIDTH)


# speedup vs baseline: 1.0006x; 1.0006x over previous
import functools

import jax
import jax.numpy as jnp
from jax import lax
from jax.experimental import pallas as pl
from jax.experimental.pallas import tpu as pltpu
from jax.experimental.pallas import tpu_sc as plsc

BT, SQ, NF = 1024, 50, 12
ROWS = BT * SQ
WIDTH = 782
NCORES, NSUB, LANES = 2, 16, 16
NW = NCORES * NSUB
RPW = ROWS // NW
RB = 16
NB = RPW // RB
GPB = RB // LANES
NBUF = 4

SEG_OFF = (0, 32, 59, 187, 315, 342, 369, 396, 524, 652)
COST_COL = (780, 781)


def _scatter_group(feat, obuf, rows, write):
    fbase = rows * NF
    obase = (rows % RB) * WIDTH
    ones = jnp.ones((LANES,), jnp.float32)
    zeros = jnp.zeros((LANES,), jnp.float32)
    for k, off in enumerate(SEG_OFF):
        col = plsc.load_gather(feat, [fbase + k]).astype(jnp.int32) + off
        plsc.store_scatter(obuf, [obase + col], ones if write else zeros)
    for j, col in enumerate(COST_COL):
        if write:
            val = plsc.load_gather(feat, [fbase + (10 + j)])
        else:
            val = zeros
        plsc.store_scatter(obuf, [obase + col], val)


def _sc_body(feat_hbm, out_hbm, feat, bufs, sems):
    wid = lax.axis_index("s") * NCORES + lax.axis_index("c")
    row0 = wid * RPW
    iota = lax.broadcasted_iota(jnp.int32, (LANES,), 0)

    pltpu.sync_copy(feat_hbm.at[pl.ds(row0 * NF, RPW * NF)], feat)
    zeros = jnp.zeros((LANES,), jnp.float32)

    def _zero(i, _):
        for b in range(NBUF):
            bufs[b][pl.ds(i * LANES, LANES)] = zeros
        return 0

    lax.fori_loop(0, RB * WIDTH // LANES, _zero, 0)

    def _out_slice(bi):
        start = pl.multiple_of((row0 + bi * RB) * WIDTH, 8)
        return out_hbm.at[pl.ds(start, RB * WIDTH)]

    def _run_batch(bi, b, first):
        if not first:
            pltpu.make_async_copy(bufs[b], _out_slice(bi), sems[b]).wait()
            for g in range(GPB):
                rows = (bi - NBUF) * RB + g * LANES + iota
                _scatter_group(feat, bufs[b], rows, write=False)
        for g in range(GPB):
            rows = bi * RB + g * LANES + iota
            _scatter_group(feat, bufs[b], rows, write=True)
        pltpu.async_copy(bufs[b], _out_slice(bi), sems[b])

    for b in range(NBUF):
        _run_batch(b, b, True)

    def _round(i, _):
        for b in range(NBUF):
            _run_batch(i * NBUF + b, b, False)
        return 0

    lax.fori_loop(1, NB // NBUF, _round, 0)

    for b in range(NBUF):
        pltpu.make_async_copy(bufs[b], _out_slice(NB - NBUF + b), sems[b]).wait()


@jax.jit
def _onehot_embed(feature_flat):
    mesh = plsc.VectorSubcoreMesh(
        core_axis_name="c", subcore_axis_name="s",
        num_cores=NCORES, num_subcores=NSUB)
    run = pl.kernel(
        _sc_body,
        out_type=jax.ShapeDtypeStruct((ROWS * WIDTH,), jnp.float32),
        mesh=mesh,
        compiler_params=pltpu.CompilerParams(needs_layout_passes=False),
        scratch_types=[
            pltpu.VMEM((RPW * NF,), jnp.float32),
            [pltpu.VMEM((RB * WIDTH,), jnp.float32) for _ in range(NBUF)],
            [pltpu.SemaphoreType.DMA for _ in range(NBUF)],
        ],
    )
    return run(feature_flat)


def kernel(feature):
    out = _onehot_embed(feature.reshape(ROWS * NF))
    return out.reshape(BT, SQ, WIDTH)


# speedup vs baseline: 1.8162x; 1.8150x over previous
import functools

import jax
import jax.numpy as jnp
from jax import lax
from jax.experimental import pallas as pl
from jax.experimental.pallas import tpu as pltpu

BT, SQ, NF = 1024, 50, 12
ROWS = BT * SQ
WIDTH = 782
RB = 512
NBLK = ROWS // RB

SEG_OFF = (0.0, 32.0, 59.0, 187.0, 315.0, 342.0, 369.0, 396.0, 524.0, 652.0)
COST_COL = (780.0, 781.0)


def _block(feat_ref, out_ref):
    col = lax.broadcasted_iota(jnp.int32, (RB, WIDTH), 1)
    acc = jnp.zeros((RB, WIDTH), jnp.float32)
    one = jnp.float32(1.0)
    for k, off in enumerate(SEG_OFF):
        tgt = feat_ref[:, k : k + 1].astype(jnp.int32) + int(off)
        acc = jnp.where(col == tgt, one, acc)
    for j, coff in enumerate(COST_COL):
        acc = jnp.where(col == int(coff), feat_ref[:, 10 + j : 11 + j], acc)
    out_ref[...] = acc


@jax.jit
def _onehot_embed(feature2d):
    return pl.pallas_call(
        _block,
        grid=(NBLK,),
        in_specs=[pl.BlockSpec((RB, NF), lambda i: (i, 0))],
        out_specs=pl.BlockSpec((RB, WIDTH), lambda i: (i, 0)),
        out_shape=jax.ShapeDtypeStruct((ROWS, WIDTH), jnp.float32),
        compiler_params=pltpu.CompilerParams(
            dimension_semantics=("arbitrary",),
        ),
    )(feature2d)


def kernel(feature):
    out = _onehot_embed(feature.reshape(ROWS, NF))
    return out.reshape(BT, SQ, WIDTH)


# speedup vs baseline: 2.3656x; 1.3025x over previous
import jax
import jax.numpy as jnp
from jax import lax
from jax.experimental import pallas as pl
from jax.experimental.pallas import tpu as pltpu

BT, SQ, NF = 1024, 50, 12
WIDTH = 782
BB = 16
NBLK = BT // BB

SEG_OFF = (0, 32, 59, 187, 315, 342, 369, 396, 524, 652)
COST_COL = (780, 781)


def _block(feat_ref, out_ref):
    col = lax.broadcasted_iota(jnp.int32, (BB, SQ, WIDTH), 2)
    acc = jnp.zeros((BB, SQ, WIDTH), jnp.float32)
    one = jnp.float32(1.0)
    for k, off in enumerate(SEG_OFF):
        tgt = feat_ref[:, :, k : k + 1].astype(jnp.int32) + off
        acc = jnp.where(col == tgt, one, acc)
    for j, coff in enumerate(COST_COL):
        acc = jnp.where(col == coff, feat_ref[:, :, 10 + j : 11 + j], acc)
    out_ref[...] = acc


@jax.jit
def kernel(feature):
    return pl.pallas_call(
        _block,
        grid=(NBLK,),
        in_specs=[pl.BlockSpec((BB, SQ, NF), lambda i: (i, 0, 0))],
        out_specs=pl.BlockSpec((BB, SQ, WIDTH), lambda i: (i, 0, 0)),
        out_shape=jax.ShapeDtypeStruct((BT, SQ, WIDTH), jnp.float32),
        compiler_params=pltpu.CompilerParams(
            dimension_semantics=("arbitrary",),
        ),
    )(feature)


# speedup vs baseline: 2.8309x; 1.1967x over previous
import jax
import jax.numpy as jnp
from jax import lax
from jax.experimental import pallas as pl
from jax.experimental.pallas import tpu as pltpu

BT, SQ, NF = 1024, 50, 12
WIDTH = 782
BB = 16
NBLK = BT // BB

SEG_OFF = (0, 32, 59, 187, 315, 342, 369, 396, 524, 652)
COST_COL = (780, 781)


def _block(feat_ref, out_ref):
    bounds = list(SEG_OFF[1:]) + [COST_COL[0], COST_COL[1]]
    col = lax.broadcasted_iota(jnp.int32, (1, 1, WIDTH), 2)
    field = sum((col >= b).astype(jnp.int32) for b in bounds)
    base = jnp.zeros((1, 1, WIDTH), jnp.int32)
    for off in bounds:
        base = jnp.where(col >= off, off, base)
    relf = (col - base).astype(jnp.float32)
    is_cost = col >= COST_COL[0]

    krow = lax.broadcasted_iota(jnp.int32, (NF, WIDTH), 0)
    m = (krow == field.reshape(1, WIDTH)).astype(jnp.float32)
    fb = lax.dot_general(
        feat_ref[...], m, (((2,), (0,)), ((), ())),
        preferred_element_type=jnp.float32,
    )
    onehot = jnp.where(fb == relf, jnp.float32(1.0), jnp.float32(0.0))
    out_ref[...] = jnp.where(is_cost, fb, onehot)


@jax.jit
def kernel(feature):
    return pl.pallas_call(
        _block,
        grid=(NBLK,),
        in_specs=[pl.BlockSpec((BB, SQ, NF), lambda i: (i, 0, 0))],
        out_specs=pl.BlockSpec((BB, SQ, WIDTH), lambda i: (i, 0, 0)),
        out_shape=jax.ShapeDtypeStruct((BT, SQ, WIDTH), jnp.float32),
        compiler_params=pltpu.CompilerParams(
            dimension_semantics=("arbitrary",),
        ),
    )(feature)


# speedup vs baseline: 3.0379x; 1.0731x over previous
import jax
import jax.numpy as jnp
from jax import lax
from jax.experimental import pallas as pl
from jax.experimental.pallas import tpu as pltpu

BT, SQ, NF = 1024, 50, 12
WIDTH = 782
BB = 64
NBLK = BT // BB

SEG_OFF = (0, 32, 59, 187, 315, 342, 369, 396, 524, 652)
COST_COL = (780, 781)


def _block(feat_ref, out_ref):
    bounds = list(SEG_OFF[1:]) + [COST_COL[0], COST_COL[1]]
    col = lax.broadcasted_iota(jnp.int32, (1, 1, WIDTH), 2)
    field = sum((col >= b).astype(jnp.int32) for b in bounds)
    base = jnp.zeros((1, 1, WIDTH), jnp.int32)
    for off in bounds:
        base = jnp.where(col >= off, off, base)
    relf = (col - base).astype(jnp.float32)
    is_cost = col >= COST_COL[0]

    krow = lax.broadcasted_iota(jnp.int32, (NF, WIDTH), 0)
    m = (krow == field.reshape(1, WIDTH)).astype(jnp.float32)
    fb = lax.dot_general(
        feat_ref[...], m, (((2,), (0,)), ((), ())),
        preferred_element_type=jnp.float32,
    )
    onehot = jnp.where(fb == relf, jnp.float32(1.0), jnp.float32(0.0))
    out_ref[...] = jnp.where(is_cost, fb, onehot)


@jax.jit
def kernel(feature):
    return pl.pallas_call(
        _block,
        grid=(NBLK,),
        in_specs=[pl.BlockSpec((BB, SQ, NF), lambda i: (i, 0, 0))],
        out_specs=pl.BlockSpec((BB, SQ, WIDTH), lambda i: (i, 0, 0)),
        out_shape=jax.ShapeDtypeStruct((BT, SQ, WIDTH), jnp.float32),
        compiler_params=pltpu.CompilerParams(
            dimension_semantics=("arbitrary",),
        ),
    )(feature)
